# baseline (speedup 1.0000x reference)
import numpy as np

import jax
import jax.numpy as jnp
from jax.experimental import pallas as pl
from jax.experimental.pallas import tpu as pltpu

_B = 128
_C = 3
_H = 224
_W = 224
_INNER = _C * _H * _W
_BB = 4
_NBLK = _B // _BB

_KS0 = np.uint32(0)
_KS1 = np.uint32(42)
_KS2 = np.uint32(0x1BD11BDA) ^ _KS0 ^ _KS1

_LO = np.float32(np.nextafter(np.float32(-1.0), np.float32(0.0)))
_SCALE = np.float32(1.0) - _LO
_SQRT2 = np.float32(np.sqrt(2.0))

_ROT_A = (13, 15, 26, 6)
_ROT_B = (17, 29, 16, 24)


def _rotl(x, d):
    return (x << np.uint32(d)) | (x >> np.uint32(32 - d))


def _threefry_bits(x1_init):
    ks = (_KS0, _KS1, _KS2)
    x1 = x1_init
    x0 = x1
    x1 = _rotl(x1, _ROT_A[0]) ^ x0
    first = True
    for i in range(5):
        rots = _ROT_A if i % 2 == 0 else _ROT_B
        for r in rots[1:] if first else rots:
            x0 = x0 + x1
            x1 = _rotl(x1, r) ^ x0
        first = False
        x0 = x0 + ks[(i + 1) % 3]
        x1 = x1 + ks[(i + 2) % 3] + np.uint32(i + 1)
    return x0 ^ x1


_P_LT = (2.81022636e-08, 3.43273939e-07, -3.5233877e-06, -4.39150654e-06,
         0.00021858087, -0.00125372503, -0.00417768164, 0.246640727,
         1.50140941)
_P_GE = (-0.000200214257, 0.000100950558, 0.00134934322, -0.00367342844,
         0.00573950773, -0.0076224613, 0.00943887047, 1.00167406,
         2.83297682)


def _erfinv(x):
    w = -jnp.log1p(-(x * x))
    lt = w < np.float32(5.0)
    wa = jnp.where(lt, w - np.float32(2.5), jnp.sqrt(w) - np.float32(3.0))
    p = jnp.where(lt, np.float32(_P_LT[0]), np.float32(_P_GE[0]))
    for ca, cb in zip(_P_LT[1:], _P_GE[1:]):
        p = jnp.where(lt, np.float32(ca), np.float32(cb)) + p * wa
    return p * x


def _noise_kernel(t_ref, tab1_ref, tab2_ref, x0_ref, xt_ref, noise_ref):
    g = pl.program_id(0)

    base = g * (_BB * _INNER) + 42
    shp = (_BB, _C, _H, _W)
    b_io = jax.lax.broadcasted_iota(jnp.int32, shp, 0)
    ch_io = jax.lax.broadcasted_iota(jnp.int32, shp, 1)
    r_io = jax.lax.broadcasted_iota(jnp.int32, shp, 2)
    c_io = jax.lax.broadcasted_iota(jnp.int32, shp, 3)
    x1_init = (base + b_io * _INNER + ch_io * (_H * _W) + r_io * _W
               + c_io).astype(jnp.uint32)

    bits = _threefry_bits(x1_init)
    fbits = (bits >> np.uint32(9)) | np.uint32(0x3F800000)
    f = jax.lax.bitcast_convert_type(fbits, jnp.float32) - np.float32(1.0)
    u = jnp.maximum(_LO, f * _SCALE + _LO)
    noise = _SQRT2 * _erfinv(u)

    noise_ref[...] = noise
    for i in range(_BB):
        tb = t_ref[g * _BB + i]
        c1 = tab1_ref[tb]
        c2 = tab2_ref[tb]
        xt_ref[i] = c1 * x0_ref[i] + c2 * noise[i]


def kernel(x_0, t, sqrt_alphas_bar, sqrt_one_minus_alphas_bar):
    grid = (_NBLK,)
    blk = pl.BlockSpec((_BB, _C, _H, _W), lambda g: (g, 0, 0, 0))
    smem = pl.BlockSpec(memory_space=pltpu.SMEM)
    xt, noise = pl.pallas_call(
        _noise_kernel,
        grid=grid,
        in_specs=[smem, smem, smem, blk],
        out_specs=[blk, blk],
        out_shape=[
            jax.ShapeDtypeStruct((_B, _C, _H, _W), jnp.float32),
            jax.ShapeDtypeStruct((_B, _C, _H, _W), jnp.float32),
        ],
        compiler_params=pltpu.CompilerParams(
            dimension_semantics=("arbitrary",)),
    )(t, sqrt_alphas_bar, sqrt_one_minus_alphas_bar, x_0)
    return (xt, noise)


# speedup vs baseline: 1.0451x; 1.0451x over previous
import numpy as np

import jax
import jax.numpy as jnp
from jax.experimental import pallas as pl
from jax.experimental.pallas import tpu as pltpu

_B = 128
_C = 3
_H = 224
_W = 224
_INNER = _C * _H * _W
_BB = 4
_NBLK = _B // _BB

_KS0 = np.uint32(0)
_KS1 = np.uint32(42)
_KS2 = np.uint32(0x1BD11BDA) ^ _KS0 ^ _KS1

_LO = np.float32(np.nextafter(np.float32(-1.0), np.float32(0.0)))
_SCALE = np.float32(1.0) - _LO
_SQRT2 = np.float32(np.sqrt(2.0))

_ROT_A = (13, 15, 26, 6)
_ROT_B = (17, 29, 16, 24)


def _rotl(x, d):
    return (x << np.uint32(d)) | (x >> np.uint32(32 - d))


def _threefry_bits(x1_init):
    ks = (_KS0, _KS1, _KS2)
    x1 = x1_init
    x0 = x1
    x1 = _rotl(x1, _ROT_A[0]) ^ x0
    first = True
    for i in range(5):
        rots = _ROT_A if i % 2 == 0 else _ROT_B
        for r in rots[1:] if first else rots:
            x0 = x0 + x1
            x1 = _rotl(x1, r) ^ x0
        first = False
        x0 = x0 + ks[(i + 1) % 3]
        x1 = x1 + ks[(i + 2) % 3] + np.uint32(i + 1)
    return x0 ^ x1


_P_LT = (2.81022636e-08, 3.43273939e-07, -3.5233877e-06, -4.39150654e-06,
         0.00021858087, -0.00125372503, -0.00417768164, 0.246640727,
         1.50140941)
_P_GE = (-0.000200214257, 0.000100950558, 0.00134934322, -0.00367342844,
         0.00573950773, -0.0076224613, 0.00943887047, 1.00167406,
         2.83297682)


def _erfinv(x):
    w = -jnp.log1p(-(x * x))
    lt = w < np.float32(5.0)
    wa = jnp.where(lt, w - np.float32(2.5), jnp.sqrt(w) - np.float32(3.0))
    p = jnp.where(lt, np.float32(_P_LT[0]), np.float32(_P_GE[0]))
    for ca, cb in zip(_P_LT[1:], _P_GE[1:]):
        p = jnp.where(lt, np.float32(ca), np.float32(cb)) + p * wa
    return p * x


def _noise_kernel(t_ref, tab1_ref, tab2_ref, x0_ref, xt_ref, noise_ref):
    g = pl.program_id(0)

    base = g * (_BB * _INNER) + 42
    shp = (_BB, _C * _H, _W)
    b_io = jax.lax.broadcasted_iota(jnp.int32, shp, 0)
    r_io = jax.lax.broadcasted_iota(jnp.int32, shp, 1)
    c_io = jax.lax.broadcasted_iota(jnp.int32, shp, 2)
    x1_init = (base + b_io * _INNER + r_io * _W + c_io).astype(jnp.uint32)

    bits = _threefry_bits(x1_init)
    fbits = (bits >> np.uint32(9)) | np.uint32(0x3F800000)
    f = jax.lax.bitcast_convert_type(fbits, jnp.float32) - np.float32(1.0)
    u = jnp.maximum(_LO, f * _SCALE + _LO)
    noise = _SQRT2 * _erfinv(u)

    noise_ref[...] = noise
    for i in range(_BB):
        tb = t_ref[g * _BB + i]
        c1 = tab1_ref[tb]
        c2 = tab2_ref[tb]
        xt_ref[i] = c1 * x0_ref[i] + c2 * noise[i]


def kernel(x_0, t, sqrt_alphas_bar, sqrt_one_minus_alphas_bar):
    x0m = x_0.reshape(_B, _C * _H, _W)
    grid = (_NBLK,)
    blk = pl.BlockSpec((_BB, _C * _H, _W), lambda g: (g, 0, 0))
    smem = pl.BlockSpec(memory_space=pltpu.SMEM)
    xt, noise = pl.pallas_call(
        _noise_kernel,
        grid=grid,
        in_specs=[smem, smem, smem, blk],
        out_specs=[blk, blk],
        out_shape=[
            jax.ShapeDtypeStruct((_B, _C * _H, _W), jnp.float32),
            jax.ShapeDtypeStruct((_B, _C * _H, _W), jnp.float32),
        ],
        compiler_params=pltpu.CompilerParams(
            dimension_semantics=("arbitrary",)),
    )(t, sqrt_alphas_bar, sqrt_one_minus_alphas_bar, x0m)
    return (xt.reshape(x_0.shape), noise.reshape(x_0.shape))


# speedup vs baseline: 1.1398x; 1.0906x over previous
import numpy as np

import jax
import jax.numpy as jnp
from jax.experimental import pallas as pl
from jax.experimental.pallas import tpu as pltpu

_B = 128
_INNER = 3 * 224 * 224
_LANES = 128
_ROWS = _INNER // _LANES
_BB = 4
_NBLK = _B // _BB

_KS0 = np.uint32(0)
_KS1 = np.uint32(42)
_KS2 = np.uint32(0x1BD11BDA) ^ _KS0 ^ _KS1

_LO = np.float32(np.nextafter(np.float32(-1.0), np.float32(0.0)))
_SCALE = np.float32(1.0) - _LO
_SQRT2 = np.float32(np.sqrt(2.0))

_ROT_A = (13, 15, 26, 6)
_ROT_B = (17, 29, 16, 24)


def _rotl(x, d):
    return (x << np.uint32(d)) | (x >> np.uint32(32 - d))


def _threefry_bits(x1_init):
    ks = (_KS0, _KS1, _KS2)
    x1 = x1_init
    x0 = x1
    x1 = _rotl(x1, _ROT_A[0]) ^ x0
    first = True
    for i in range(5):
        rots = _ROT_A if i % 2 == 0 else _ROT_B
        for r in rots[1:] if first else rots:
            x0 = x0 + x1
            x1 = _rotl(x1, r) ^ x0
        first = False
        x0 = x0 + ks[(i + 1) % 3]
        x1 = x1 + ks[(i + 2) % 3] + np.uint32(i + 1)
    return x0 ^ x1


_P_LT = (2.81022636e-08, 3.43273939e-07, -3.5233877e-06, -4.39150654e-06,
         0.00021858087, -0.00125372503, -0.00417768164, 0.246640727,
         1.50140941)
_P_GE = (-0.000200214257, 0.000100950558, 0.00134934322, -0.00367342844,
         0.00573950773, -0.0076224613, 0.00943887047, 1.00167406,
         2.83297682)


def _erfinv(x):
    w = -jnp.log1p(-(x * x))
    lt = w < np.float32(5.0)
    wa = jnp.where(lt, w - np.float32(2.5), jnp.sqrt(w) - np.float32(3.0))
    p = jnp.where(lt, np.float32(_P_LT[0]), np.float32(_P_GE[0]))
    for ca, cb in zip(_P_LT[1:], _P_GE[1:]):
        p = jnp.where(lt, np.float32(ca), np.float32(cb)) + p * wa
    return p * x


def _noise_kernel(t_ref, tab1_ref, tab2_ref, x0_ref, xt_ref, noise_ref):
    g = pl.program_id(0)

    base = g * (_BB * _INNER) + 42
    shp = (_BB, _ROWS, _LANES)
    b_io = jax.lax.broadcasted_iota(jnp.int32, shp, 0)
    r_io = jax.lax.broadcasted_iota(jnp.int32, shp, 1)
    c_io = jax.lax.broadcasted_iota(jnp.int32, shp, 2)
    x1_init = (base + b_io * _INNER + r_io * _LANES + c_io).astype(jnp.uint32)

    bits = _threefry_bits(x1_init)
    fbits = (bits >> np.uint32(9)) | np.uint32(0x3F800000)
    f = jax.lax.bitcast_convert_type(fbits, jnp.float32) - np.float32(1.0)
    u = jnp.maximum(_LO, f * _SCALE + _LO)
    noise = _SQRT2 * _erfinv(u)

    noise_ref[...] = noise
    for i in range(_BB):
        tb = t_ref[g * _BB + i]
        c1 = tab1_ref[tb]
        c2 = tab2_ref[tb]
        xt_ref[i] = c1 * x0_ref[i] + c2 * noise[i]


def kernel(x_0, t, sqrt_alphas_bar, sqrt_one_minus_alphas_bar):
    x0r = x_0.reshape(_B, _ROWS, _LANES)
    grid = (_NBLK,)
    blk = pl.BlockSpec((_BB, _ROWS, _LANES), lambda g: (g, 0, 0))
    smem = pl.BlockSpec(memory_space=pltpu.SMEM)
    xt, noise = pl.pallas_call(
        _noise_kernel,
        grid=grid,
        in_specs=[smem, smem, smem, blk],
        out_specs=[blk, blk],
        out_shape=[
            jax.ShapeDtypeStruct((_B, _ROWS, _LANES), jnp.float32),
            jax.ShapeDtypeStruct((_B, _ROWS, _LANES), jnp.float32),
        ],
        compiler_params=pltpu.CompilerParams(
            dimension_semantics=("arbitrary",)),
    )(t, sqrt_alphas_bar, sqrt_one_minus_alphas_bar, x0r)
    return (xt.reshape(x_0.shape), noise.reshape(x_0.shape))


# speedup vs baseline: 1.1657x; 1.0227x over previous
import numpy as np

import jax
import jax.numpy as jnp
from jax.experimental import pallas as pl
from jax.experimental.pallas import tpu as pltpu

_B = 128
_INNER = 3 * 224 * 224
_LANES = 128
_ROWS = _INNER // _LANES
_BB = 4
_NBLK = _B // _BB

_KS0 = np.uint32(0)
_KS1 = np.uint32(42)
_KS2 = np.uint32(0x1BD11BDA) ^ _KS0 ^ _KS1

_LO = np.float32(np.nextafter(np.float32(-1.0), np.float32(0.0)))
_SCALE = np.float32(1.0) - _LO
_SQRT2 = np.float32(np.sqrt(2.0))

_ROT_A = (13, 15, 26, 6)
_ROT_B = (17, 29, 16, 24)


def _rotl(x, d):
    return (x << np.uint32(d)) | (x >> np.uint32(32 - d))


def _threefry_bits(x1_init):
    ks = (_KS0, _KS1, _KS2)
    x1 = x1_init
    x0 = x1
    x1 = _rotl(x1, _ROT_A[0]) ^ x0
    first = True
    for i in range(5):
        rots = _ROT_A if i % 2 == 0 else _ROT_B
        for r in rots[1:] if first else rots:
            x0 = x0 + x1
            x1 = _rotl(x1, r) ^ x0
        first = False
        x0 = x0 + ks[(i + 1) % 3]
        x1 = x1 + ks[(i + 2) % 3] + np.uint32(i + 1)
    return x0 ^ x1


_P_S = (4.2129054e-06, -0.0001386443, 0.0017952744, -0.012282859,
        0.04905399, -0.117964946, 0.1728634, -0.16588892, 0.11305284,
        -0.033115488, 0.33410347, -0.00045962588, 1.2533227)


def _sqrt2_erfinv(x):
    sw = jnp.sqrt(-jnp.log1p(-(x * x)))
    p = np.float32(_P_S[0])
    for c in _P_S[1:]:
        p = np.float32(c) + p * sw
    return p * x


def _noise_kernel(t_ref, tab1_ref, tab2_ref, x0_ref, xt_ref, noise_ref):
    g = pl.program_id(0)

    base = g * (_BB * _INNER) + 42
    shp = (_BB, _ROWS, _LANES)
    b_io = jax.lax.broadcasted_iota(jnp.int32, shp, 0)
    r_io = jax.lax.broadcasted_iota(jnp.int32, shp, 1)
    c_io = jax.lax.broadcasted_iota(jnp.int32, shp, 2)
    x1_init = (base + b_io * _INNER + r_io * _LANES + c_io).astype(jnp.uint32)

    bits = _threefry_bits(x1_init)
    fbits = (bits >> np.uint32(9)) | np.uint32(0x3F800000)
    f = jax.lax.bitcast_convert_type(fbits, jnp.float32) - np.float32(1.0)
    u = jnp.maximum(_LO, f * _SCALE + _LO)
    noise = _sqrt2_erfinv(u)

    noise_ref[...] = noise
    for i in range(_BB):
        tb = t_ref[g * _BB + i]
        c1 = tab1_ref[tb]
        c2 = tab2_ref[tb]
        xt_ref[i] = c1 * x0_ref[i] + c2 * noise[i]


def kernel(x_0, t, sqrt_alphas_bar, sqrt_one_minus_alphas_bar):
    x0r = x_0.reshape(_B, _ROWS, _LANES)
    grid = (_NBLK,)
    blk = pl.BlockSpec((_BB, _ROWS, _LANES), lambda g: (g, 0, 0))
    smem = pl.BlockSpec(memory_space=pltpu.SMEM)
    xt, noise = pl.pallas_call(
        _noise_kernel,
        grid=grid,
        in_specs=[smem, smem, smem, blk],
        out_specs=[blk, blk],
        out_shape=[
            jax.ShapeDtypeStruct((_B, _ROWS, _LANES), jnp.float32),
            jax.ShapeDtypeStruct((_B, _ROWS, _LANES), jnp.float32),
        ],
        compiler_params=pltpu.CompilerParams(
            dimension_semantics=("arbitrary",)),
    )(t, sqrt_alphas_bar, sqrt_one_minus_alphas_bar, x0r)
    return (xt.reshape(x_0.shape), noise.reshape(x_0.shape))
